# Initial kernel scaffold; baseline (speedup 1.0000x reference)
import functools

import jax
import jax.numpy as jnp
from jax import lax
from jax.experimental import pallas as pl
from jax.experimental.pallas import tpu as pltpu
from jax.experimental.pallas import tpu_sc as plsc

NC = 2
NS = 16
NW = NC * NS
L = 16
CH = 128

_HIGH = lax.Precision.HIGHEST


def _mesh():
    return plsc.VectorSubcoreMesh(core_axis_name="c", subcore_axis_name="s")


def _deg_sc(edge_index, nn):
    E = edge_index.shape[1]
    EW = E // NW
    n_full = EW // CH
    tail = EW - n_full * CH
    RT = nn // NS
    ZR = 125
    assert E % NW == 0 and nn % NS == 0 and RT % ZR == 0 and tail % 8 == 0

    def body(edge_hbm, out_hbm, dsti, dstt, ones_v, zero_v, acc_sh):
        cid = lax.axis_index("c")
        sid = lax.axis_index("s")
        wid = sid * NC + cid
        base = wid * EW

        @pl.loop(0, ZR)
        def _(r):
            zero_v[r, pl.ds(0, L)] = jnp.zeros((L,), jnp.float32)

        @pl.loop(0, CH)
        def _(r):
            ones_v[r, pl.ds(0, L)] = jnp.ones((L,), jnp.float32)

        @pl.loop(0, RT, step=ZR)
        def _(r):
            pltpu.sync_copy(zero_v, acc_sh.at[pl.ds(sid * RT + r, ZR)])

        plsc.subcore_barrier()

        @pl.loop(0, n_full * CH, step=CH)
        def _(j):
            pltpu.sync_copy(edge_hbm.at[1, pl.ds(base + j, CH)], dsti.at[0])
            pltpu.sync_copy(ones_v, acc_sh.at[dsti.at[0]], add=True)

        if tail:
            pltpu.sync_copy(
                edge_hbm.at[1, pl.ds(base + n_full * CH, tail)], dstt.at[0])
            pltpu.sync_copy(ones_v.at[pl.ds(0, tail)],
                            acc_sh.at[dstt.at[0]], add=True)

        plsc.subcore_barrier()

        @pl.loop(0, RT, step=ZR)
        def _(r):
            pltpu.sync_copy(acc_sh.at[pl.ds(sid * RT + r, ZR)],
                            out_hbm.at[cid, pl.ds(sid * RT + r, ZR)])

    k = pl.kernel(
        body,
        out_type=jax.ShapeDtypeStruct((NC, nn, L), jnp.float32),
        mesh=_mesh(),
        scratch_types=[
            pltpu.VMEM((1, CH), jnp.int32),
            pltpu.VMEM((1, tail if tail else 8), jnp.int32),
            pltpu.VMEM((CH, L), jnp.float32),
            pltpu.VMEM((ZR, L), jnp.float32),
            pltpu.VMEM_SHARED((nn, L), jnp.float32),
        ],
    )
    return k(edge_index)


def _agg_sc(hn, edge_index, norm16, with_w):
    nn, D = hn.shape
    E = edge_index.shape[1]
    EW = E // NW
    n_full = EW // CH
    tail = EW - n_full * CH
    RT = nn // NS
    ZR = 125
    assert E % NW == 0 and nn % NS == 0 and RT % ZR == 0 and tail % 8 == 0

    def body(*refs):
        if with_w:
            (hn_hbm, edge_hbm, n16_hbm, agg_hbm, w_hbm,
             srci, dsti, srct, dstt, rows, rowst, n16r, n16rt,
             zero_v, z16_v, agg_sh, w_sh) = refs
        else:
            (hn_hbm, edge_hbm, agg_hbm,
             srci, dsti, srct, dstt, rows, rowst,
             zero_v, agg_sh) = refs
        cid = lax.axis_index("c")
        sid = lax.axis_index("s")
        wid = sid * NC + cid
        base = wid * EW

        @pl.loop(0, ZR)
        def _(r):
            @pl.loop(0, D, step=L)
            def _(j):
                zero_v[r, pl.ds(j, L)] = jnp.zeros((L,), jnp.float32)

        @pl.loop(0, RT, step=ZR)
        def _(r):
            pltpu.sync_copy(zero_v, agg_sh.at[pl.ds(sid * RT + r, ZR)])

        if with_w:
            @pl.loop(0, ZR)
            def _(r):
                z16_v[r, pl.ds(0, L)] = jnp.zeros((L,), jnp.float32)

            @pl.loop(0, RT, step=ZR)
            def _(r):
                pltpu.sync_copy(z16_v, w_sh.at[pl.ds(sid * RT + r, ZR)])

        plsc.subcore_barrier()

        def chunk(off, cc, s_v, d_v, r_v, nr_v):
            pltpu.sync_copy(edge_hbm.at[0, pl.ds(off, cc)], s_v.at[0])
            pltpu.sync_copy(edge_hbm.at[1, pl.ds(off, cc)], d_v.at[0])
            pltpu.sync_copy(hn_hbm.at[s_v.at[0]], r_v)
            pltpu.sync_copy(r_v, agg_sh.at[d_v.at[0]], add=True)
            if with_w:
                pltpu.sync_copy(n16_hbm.at[d_v.at[0]], nr_v)
                pltpu.sync_copy(nr_v, w_sh.at[s_v.at[0]], add=True)

        @pl.loop(0, n_full * CH, step=CH)
        def _(j):
            chunk(base + j, CH, srci, dsti, rows,
                  n16r if with_w else None)

        if tail:
            chunk(base + n_full * CH, tail, srct, dstt, rowst,
                  n16rt if with_w else None)

        plsc.subcore_barrier()

        @pl.loop(0, RT, step=ZR)
        def _(r):
            pltpu.sync_copy(agg_sh.at[pl.ds(sid * RT + r, ZR)],
                            agg_hbm.at[cid, pl.ds(sid * RT + r, ZR)])

        if with_w:
            @pl.loop(0, RT, step=ZR)
            def _(r):
                pltpu.sync_copy(w_sh.at[pl.ds(sid * RT + r, ZR)],
                                w_hbm.at[cid, pl.ds(sid * RT + r, ZR)])

    tl = tail if tail else 8
    out_type = [jax.ShapeDtypeStruct((NC, nn, D), jnp.float32)]
    scratch = [
        pltpu.VMEM((1, CH), jnp.int32),
        pltpu.VMEM((1, CH), jnp.int32),
        pltpu.VMEM((1, tl), jnp.int32),
        pltpu.VMEM((1, tl), jnp.int32),
        pltpu.VMEM((CH, D), jnp.float32),
        pltpu.VMEM((tl, D), jnp.float32),
    ]
    if with_w:
        out_type.append(jax.ShapeDtypeStruct((NC, nn, L), jnp.float32))
        scratch += [
            pltpu.VMEM((CH, L), jnp.float32),
            pltpu.VMEM((tl, L), jnp.float32),
            pltpu.VMEM((ZR, D), jnp.float32),
            pltpu.VMEM((ZR, L), jnp.float32),
            pltpu.VMEM_SHARED((nn, D), jnp.float32),
            pltpu.VMEM_SHARED((nn, L), jnp.float32),
        ]
    else:
        scratch += [
            pltpu.VMEM((ZR, D), jnp.float32),
            pltpu.VMEM_SHARED((nn, D), jnp.float32),
        ]

    k = pl.kernel(body, out_type=out_type, mesh=_mesh(),
                  scratch_types=scratch)
    if with_w:
        return k(hn, edge_index, norm16)
    return (k(hn, edge_index),)


def _tc_prolog(deg_part, x, W_in, b_in):
    nn, D = x.shape

    def body(dp_ref, x_ref, w_ref, b_ref, h_ref, nc_ref, n16_ref):
        deg = dp_ref[0, :, 0:1] + dp_ref[1, :, 0:1]
        norm = lax.rsqrt(jnp.maximum(deg, 1.0))
        nc_ref[...] = norm
        n16_ref[...] = jnp.broadcast_to(norm, (nn, L))
        h = jnp.dot(x_ref[...], w_ref[...],
                    preferred_element_type=jnp.float32, precision=_HIGH)
        h_ref[...] = (h + b_ref[...]) * norm

    return pl.pallas_call(
        body,
        out_shape=[
            jax.ShapeDtypeStruct((nn, D), jnp.float32),
            jax.ShapeDtypeStruct((nn, 1), jnp.float32),
            jax.ShapeDtypeStruct((nn, L), jnp.float32),
        ],
    )(deg_part, x, W_in, b_in)


def _tc_mid(agg_part, norm_col, W, b):
    nn = agg_part.shape[1]
    D = agg_part.shape[2]

    def body(a_ref, nc_ref, w_ref, b_ref, o_ref):
        norm = nc_ref[...]
        agg = (a_ref[0] + a_ref[1]) * norm
        h = jnp.dot(agg, w_ref[...],
                    preferred_element_type=jnp.float32, precision=_HIGH)
        o_ref[...] = jnp.maximum(h + b_ref[...], 0.0) * norm

    return pl.pallas_call(
        body, out_shape=jax.ShapeDtypeStruct((nn, D), jnp.float32),
    )(agg_part, norm_col, W, b)


def _tc_final(agg_part, norm_col, w_part, W1, b1, W2, b2, W_out, b_out):
    nn = agg_part.shape[1]
    D = agg_part.shape[2]

    def body(a_ref, nc_ref, wp_ref, w1_ref, b1_ref, w2_ref, b2_ref,
             wo_ref, bo_ref, o_ref):
        norm = nc_ref[...]
        agg = (a_ref[0] + a_ref[1]) * norm
        hc = jnp.maximum(
            jnp.dot(agg, w1_ref[...],
                    preferred_element_type=jnp.float32, precision=_HIGH)
            + b1_ref[...], 0.0)
        w_col = wp_ref[0, :, 0:1] + wp_ref[1, :, 0:1]
        v = jnp.sum(hc * (w_col * norm), axis=0, keepdims=True)
        t = jnp.dot(v, w2_ref[...],
                    preferred_element_type=jnp.float32, precision=_HIGH)
        t = t + jnp.float32(nn) * b2_ref[...]
        o_ref[...] = jnp.dot(t, wo_ref[...],
                             preferred_element_type=jnp.float32,
                             precision=_HIGH) + bo_ref[...]

    return pl.pallas_call(
        body, out_shape=jax.ShapeDtypeStruct((1, D), jnp.float32),
    )(agg_part, norm_col, w_part, W1, b1, W2, b2, W_out, b_out)


def kernel(x, edge_index, W_in, b_in, W0, b0, W1, b1, W2, b2, W_out, b_out):
    nn, D = x.shape
    b_in2 = b_in.reshape(1, D)
    b02 = b0.reshape(1, D)
    b12 = b1.reshape(1, D)
    b22 = b2.reshape(1, D)
    b_out2 = b_out.reshape(1, D)

    deg_part = _deg_sc(edge_index, nn)
    h0n, norm_col, norm16 = _tc_prolog(deg_part, x, W_in, b_in2)
    agg0, w_part = _agg_sc(h0n, edge_index, norm16, with_w=True)
    h1n = _tc_mid(agg0, norm_col, W0, b02)
    agg1 = _agg_sc(h1n, edge_index, None, with_w=False)[0]
    return _tc_final(agg1, norm_col, w_part, W1, b12, W2, b22, W_out, b_out2)


# speedup vs baseline: 9.0659x; 9.0659x over previous
import dataclasses
import functools

import jax
import jax.numpy as jnp
from jax import lax
from jax.experimental import pallas as pl
from jax.experimental.pallas import tpu as pltpu
from jax.experimental.pallas import tpu_sc as plsc

NC = 2
NS = 16
NW = NC * NS
L = 16
CH = 128

_HIGH = lax.Precision.HIGHEST


def _mesh():
    return plsc.VectorSubcoreMesh(core_axis_name="c", subcore_axis_name="s")


def _sc_params():
    cp = pltpu.CompilerParams()
    if "needs_layout_passes" in pltpu.CompilerParams.__dataclass_fields__:
        cp = dataclasses.replace(cp, needs_layout_passes=False)
    return cp


def _pad(nn):
    return -(-nn // (NS * CH)) * (NS * CH)


def _deg_sc(dst, nn):
    E = dst.shape[0]
    EW = E // NW
    nnp = _pad(nn)
    assert E % (NW * L) == 0

    def body(dst_hbm, out_hbm, idx_v, acc_v):
        cid = lax.axis_index("c")
        sid = lax.axis_index("s")
        wid = sid * NC + cid
        pltpu.sync_copy(dst_hbm.at[pl.ds(wid * EW, EW)], idx_v)

        @pl.loop(0, nnp, step=L)
        def _(i):
            acc_v[0, pl.ds(i, L)] = jnp.zeros((L,), jnp.float32)

        ones = jnp.ones((L,), jnp.float32)
        z16 = jnp.zeros((L,), jnp.int32)

        @pl.loop(0, EW, step=L)
        def _(i):
            d16 = idx_v[pl.ds(i, L)]
            plsc.addupdate_scatter(acc_v, [z16, d16], ones)

        pltpu.sync_copy(acc_v, out_hbm.at[wid])

    k = pl.kernel(
        body,
        out_type=jax.ShapeDtypeStruct((NW, 1, nnp), jnp.float32),
        mesh=_mesh(),
        scratch_types=[
            pltpu.VMEM((EW,), jnp.int32),
            pltpu.VMEM((1, nnp), jnp.float32),
        ],
        compiler_params=_sc_params(),
    )
    return k(dst)


def _agg_sc(hn, src, dst, norm_row, with_w):
    nn, D = hn.shape
    E = src.shape[0]
    EW = E // NW
    n_full = EW // CH
    tail = EW - n_full * CH
    nnp = _pad(nn)
    RT = nnp // NS
    ZR = 32
    assert E % NW == 0 and RT % ZR == 0 and tail % L == 0 and tail % 8 == 0

    def body(*refs):
        if with_w:
            (hn_hbm, src_hbm, dst_hbm, nr_hbm, agg_hbm, w_hbm,
             srci, dsti, srct, dstt, rows, rowst,
             zero_v, agg_sh, norm_v, w_v) = refs
        else:
            (hn_hbm, src_hbm, dst_hbm, agg_hbm,
             srci, dsti, srct, dstt, rows, rowst,
             zero_v, agg_sh) = refs
        cid = lax.axis_index("c")
        sid = lax.axis_index("s")
        wid = sid * NC + cid
        base = wid * EW

        @pl.loop(0, ZR)
        def _(r):
            @pl.loop(0, D, step=L)
            def _(j):
                zero_v[r, pl.ds(j, L)] = jnp.zeros((L,), jnp.float32)

        @pl.loop(0, RT, step=ZR)
        def _(r):
            pltpu.sync_copy(zero_v, agg_sh.at[pl.ds(sid * RT + r, ZR)])

        if with_w:
            pltpu.sync_copy(nr_hbm, norm_v)

            @pl.loop(0, nnp, step=L)
            def _(i):
                w_v[0, pl.ds(i, L)] = jnp.zeros((L,), jnp.float32)

        z16 = jnp.zeros((L,), jnp.int32)
        plsc.subcore_barrier()

        def chunk(off, cc, s_v, d_v, r_v):
            pltpu.sync_copy(src_hbm.at[pl.ds(off, cc)], s_v.at[0])
            pltpu.sync_copy(dst_hbm.at[pl.ds(off, cc)], d_v.at[0])
            pltpu.sync_copy(hn_hbm.at[s_v.at[0]], r_v)
            pltpu.sync_copy(r_v, agg_sh.at[d_v.at[0]], add=True)
            if with_w:
                @pl.loop(0, cc, step=L)
                def _(k2):
                    s16 = s_v[0, pl.ds(k2, L)]
                    d16 = d_v[0, pl.ds(k2, L)]
                    vals = plsc.load_gather(norm_v, [z16, d16])
                    plsc.addupdate_scatter(w_v, [z16, s16], vals)

        @pl.loop(0, n_full * CH, step=CH)
        def _(j):
            chunk(base + j, CH, srci, dsti, rows)

        if tail:
            chunk(base + n_full * CH, tail, srct, dstt, rowst)

        plsc.subcore_barrier()

        @pl.loop(0, RT, step=ZR)
        def _(r):
            pltpu.sync_copy(agg_sh.at[pl.ds(sid * RT + r, ZR)],
                            agg_hbm.at[cid, pl.ds(sid * RT + r, ZR)])

        if with_w:
            pltpu.sync_copy(w_v, w_hbm.at[wid])

    tl = tail if tail else 8
    out_type = [jax.ShapeDtypeStruct((NC, nnp, D), jnp.float32)]
    scratch = [
        pltpu.VMEM((1, CH), jnp.int32),
        pltpu.VMEM((1, CH), jnp.int32),
        pltpu.VMEM((1, tl), jnp.int32),
        pltpu.VMEM((1, tl), jnp.int32),
        pltpu.VMEM((CH, D), jnp.float32),
        pltpu.VMEM((tl, D), jnp.float32),
        pltpu.VMEM((ZR, D), jnp.float32),
        pltpu.VMEM_SHARED((nnp, D), jnp.float32),
    ]
    if with_w:
        out_type.append(jax.ShapeDtypeStruct((NW, 1, nnp), jnp.float32))
        scratch += [
            pltpu.VMEM((1, nnp), jnp.float32),
            pltpu.VMEM((1, nnp), jnp.float32),
        ]

    k = pl.kernel(body, out_type=out_type if with_w else out_type[0],
                  mesh=_mesh(), scratch_types=scratch,
                  compiler_params=_sc_params())
    if with_w:
        return k(hn, src, dst, norm_row)
    return (k(hn, src, dst),)


def _tc_prolog(deg_part, x, W_in, b_in):
    nn, D = x.shape
    nnp = deg_part.shape[1]

    def body(dp_ref, x_ref, w_ref, b_ref, h_ref, nc_ref, nr_ref):
        deg = jnp.sum(dp_ref[...], axis=0, keepdims=True)
        norm_r = lax.rsqrt(jnp.maximum(deg, 1.0))
        nr_ref[...] = norm_r
        norm_c = jnp.transpose(norm_r)
        nc_ref[...] = norm_c
        h = jnp.dot(x_ref[...], w_ref[...],
                    preferred_element_type=jnp.float32, precision=_HIGH)
        h_ref[...] = (h + b_ref[...]) * norm_c[:nn]

    return pl.pallas_call(
        body,
        out_shape=[
            jax.ShapeDtypeStruct((nn, D), jnp.float32),
            jax.ShapeDtypeStruct((nnp, 1), jnp.float32),
            jax.ShapeDtypeStruct((1, nnp), jnp.float32),
        ],
    )(deg_part, x, W_in, b_in)


def _tc_mid(agg_part, norm_col, W, b, nn):
    D = agg_part.shape[2]

    def body(a_ref, nc_ref, w_ref, b_ref, o_ref):
        norm = nc_ref[...][:nn]
        a = a_ref[...]
        agg = (a[0, :nn] + a[1, :nn]) * norm
        h = jnp.dot(agg, w_ref[...],
                    preferred_element_type=jnp.float32, precision=_HIGH)
        o_ref[...] = jnp.maximum(h + b_ref[...], 0.0) * norm

    return pl.pallas_call(
        body, out_shape=jax.ShapeDtypeStruct((nn, D), jnp.float32),
    )(agg_part, norm_col, W, b)


def _tc_final(agg_part, norm_col, norm_row, w_part, W1, b1, W2, b2,
              W_out, b_out, nn):
    D = agg_part.shape[2]

    def body(a_ref, nc_ref, nr_ref, wp_ref, w1_ref, b1_ref, w2_ref, b2_ref,
             wo_ref, bo_ref, o_ref):
        norm = nc_ref[...][:nn]
        a = a_ref[...]
        agg = (a[0, :nn] + a[1, :nn]) * norm
        hc = jnp.maximum(
            jnp.dot(agg, w1_ref[...],
                    preferred_element_type=jnp.float32, precision=_HIGH)
            + b1_ref[...], 0.0)
        w_row = jnp.sum(wp_ref[...], axis=0, keepdims=True)
        c_col = jnp.transpose(w_row * nr_ref[...])[:nn]
        v = jnp.sum(hc * c_col, axis=0, keepdims=True)
        t = jnp.dot(v, w2_ref[...],
                    preferred_element_type=jnp.float32, precision=_HIGH)
        t = t + jnp.float32(nn) * b2_ref[...]
        o_ref[...] = jnp.dot(t, wo_ref[...],
                             preferred_element_type=jnp.float32,
                             precision=_HIGH) + bo_ref[...]

    return pl.pallas_call(
        body, out_shape=jax.ShapeDtypeStruct((1, D), jnp.float32),
    )(agg_part, norm_col, norm_row, w_part, W1, b1, W2, b2, W_out, b_out)


def kernel(x, edge_index, W_in, b_in, W0, b0, W1, b1, W2, b2, W_out, b_out):
    nn, D = x.shape
    nnp = _pad(nn)
    b_in2 = b_in.reshape(1, D)
    b02 = b0.reshape(1, D)
    b12 = b1.reshape(1, D)
    b22 = b2.reshape(1, D)
    b_out2 = b_out.reshape(1, D)
    src = edge_index[0]
    dst = edge_index[1]

    deg_part = _deg_sc(dst, nn).reshape(NW, nnp)
    h0n, norm_col, norm_row = _tc_prolog(deg_part, x, W_in, b_in2)
    agg0, w_part = _agg_sc(h0n, src, dst, norm_row, with_w=True)
    w_part = w_part.reshape(NW, nnp)
    h1n = _tc_mid(agg0, norm_col, W0, b02, nn)
    agg1 = _agg_sc(h1n, src, dst, None, with_w=False)[0]
    return _tc_final(agg1, norm_col, norm_row, w_part, W1, b12, W2, b22,
                     W_out, b_out2, nn)


# speedup vs baseline: 14.9168x; 1.6454x over previous
import dataclasses
import functools

import jax
import jax.numpy as jnp
from jax import lax
from jax.experimental import pallas as pl
from jax.experimental.pallas import tpu as pltpu
from jax.experimental.pallas import tpu_sc as plsc

NC = 2
NS = 16
NW = NC * NS
L = 16
CH = 128

_HIGH = lax.Precision.HIGHEST


def _mesh():
    return plsc.VectorSubcoreMesh(core_axis_name="c", subcore_axis_name="s")


def _sc_params():
    cp = pltpu.CompilerParams()
    if "needs_layout_passes" in pltpu.CompilerParams.__dataclass_fields__:
        cp = dataclasses.replace(cp, needs_layout_passes=False)
    return cp


def _pad(nn):
    return -(-nn // (NS * CH)) * (NS * CH)


def _deg_sc(dst, nn):
    E = dst.shape[0]
    EW = E // NW
    nnp = _pad(nn)
    assert E % (NW * L) == 0

    def body(dst_hbm, out_hbm, idx_v, acc_v):
        cid = lax.axis_index("c")
        sid = lax.axis_index("s")
        wid = sid * NC + cid
        pltpu.sync_copy(dst_hbm.at[pl.ds(wid * EW, EW)], idx_v)

        @pl.loop(0, nnp, step=L)
        def _(i):
            acc_v[0, pl.ds(i, L)] = jnp.zeros((L,), jnp.float32)

        ones = jnp.ones((L,), jnp.float32)
        z16 = jnp.zeros((L,), jnp.int32)

        @pl.loop(0, EW, step=L)
        def _(i):
            d16 = idx_v[pl.ds(i, L)]
            plsc.addupdate_scatter(acc_v, [z16, d16], ones)

        pltpu.sync_copy(acc_v, out_hbm.at[wid])

    k = pl.kernel(
        body,
        out_type=jax.ShapeDtypeStruct((NW, 1, nnp), jnp.float32),
        mesh=_mesh(),
        scratch_types=[
            pltpu.VMEM((EW,), jnp.int32),
            pltpu.VMEM((1, nnp), jnp.float32),
        ],
        compiler_params=_sc_params(),
    )
    return k(dst)


def _w_sc(src, dst, norm_row, nn):
    E = src.shape[0]
    EW = E // NW
    nnp = _pad(nn)
    assert E % (NW * L) == 0

    def body(src_hbm, dst_hbm, nr_hbm, out_hbm, sidx, didx, norm_v, w_v):
        cid = lax.axis_index("c")
        sid = lax.axis_index("s")
        wid = sid * NC + cid
        base = wid * EW
        pltpu.sync_copy(src_hbm.at[pl.ds(base, EW)], sidx)
        pltpu.sync_copy(dst_hbm.at[pl.ds(base, EW)], didx)
        pltpu.sync_copy(nr_hbm, norm_v)

        @pl.loop(0, nnp, step=L)
        def _(i):
            w_v[0, pl.ds(i, L)] = jnp.zeros((L,), jnp.float32)

        z16 = jnp.zeros((L,), jnp.int32)

        @pl.loop(0, EW, step=L)
        def _(i):
            s16 = sidx[pl.ds(i, L)]
            d16 = didx[pl.ds(i, L)]
            vals = plsc.load_gather(norm_v, [z16, d16])
            plsc.addupdate_scatter(w_v, [z16, s16], vals)

        pltpu.sync_copy(w_v, out_hbm.at[wid])

    k = pl.kernel(
        body,
        out_type=jax.ShapeDtypeStruct((NW, 1, nnp), jnp.float32),
        mesh=_mesh(),
        scratch_types=[
            pltpu.VMEM((EW,), jnp.int32),
            pltpu.VMEM((EW,), jnp.int32),
            pltpu.VMEM((1, nnp), jnp.float32),
            pltpu.VMEM((1, nnp), jnp.float32),
        ],
        compiler_params=_sc_params(),
    )
    return k(src, dst, norm_row)


def _agg_sc(hn, src, dst):
    nn, D = hn.shape
    E = src.shape[0]
    EW = E // NW
    n_full = EW // CH
    tail = EW - n_full * CH
    nnp = _pad(nn)
    RT = nnp // NS
    ZR = 32
    assert E % NW == 0 and RT % ZR == 0 and tail % 8 == 0
    assert n_full >= 6 and n_full % 4 == 2

    def body(hn_hbm, src_hbm, dst_hbm, agg_hbm,
             srci, dsti, srct, dstt, rows, rowst, zero_v, agg_sh,
             semi0, semi1, semi2, semi3, semg0, semg1, sems0, sems1):
        semi = (semi0, semi1, semi2, semi3)
        semg = (semg0, semg1)
        sems = (sems0, sems1)
        cid = lax.axis_index("c")
        sid = lax.axis_index("s")
        wid = sid * NC + cid
        base = wid * EW

        @pl.loop(0, ZR)
        def _(r):
            @pl.loop(0, D, step=L)
            def _(j):
                zero_v[r, pl.ds(j, L)] = jnp.zeros((L,), jnp.float32)

        @pl.loop(0, RT, step=ZR)
        def _(r):
            pltpu.sync_copy(zero_v, agg_sh.at[pl.ds(sid * RT + r, ZR)])

        plsc.subcore_barrier()

        def idx_issue(c, s):
            off = base + c * CH
            pltpu.async_copy(src_hbm.at[pl.ds(off, CH)], srci.at[s], semi[s])
            pltpu.async_copy(dst_hbm.at[pl.ds(off, CH)], dsti.at[s], semi[s])

        def idx_wait(s):
            pltpu.make_async_copy(
                src_hbm.at[pl.ds(0, CH)], srci.at[s], semi[s]).wait()
            pltpu.make_async_copy(
                dst_hbm.at[pl.ds(0, CH)], dsti.at[s], semi[s]).wait()

        def sct_wait(s2):
            pltpu.make_async_copy(
                rows.at[s2], agg_sh.at[dsti.at[0]], sems[s2]).wait()

        def step(c, s4, s2, first, prefetch):
            if not first:
                sct_wait(s2)
            idx_wait(s4)
            pltpu.async_copy(
                hn_hbm.at[srci.at[s4]], rows.at[s2], semg[s2])
            pltpu.make_async_copy(
                hn_hbm.at[srci.at[0]], rows.at[s2], semg[s2]).wait()
            pltpu.async_copy(
                rows.at[s2], agg_sh.at[dsti.at[s4]], sems[s2], add=True)
            if prefetch:
                idx_issue(c + 2, (s4 + 2) % 4)

        idx_issue(0, 0)
        idx_issue(1, 1)
        for k4 in range(4):
            step(k4, k4, k4 % 2, k4 < 2, True)

        @pl.loop(4, n_full - 2, step=4)
        def _(g):
            for k4 in range(4):
                step(g + k4, k4, k4 % 2, False, True)

        step(n_full - 2, 0, 0, False, False)
        step(n_full - 1, 1, 1, False, False)
        sct_wait(0)
        sct_wait(1)

        if tail:
            off = base + n_full * CH
            pltpu.sync_copy(src_hbm.at[pl.ds(off, tail)], srct.at[0])
            pltpu.sync_copy(dst_hbm.at[pl.ds(off, tail)], dstt.at[0])
            pltpu.sync_copy(hn_hbm.at[srct.at[0]], rowst)
            pltpu.sync_copy(rowst, agg_sh.at[dstt.at[0]], add=True)

        plsc.subcore_barrier()

        @pl.loop(0, RT, step=ZR)
        def _(r):
            pltpu.sync_copy(agg_sh.at[pl.ds(sid * RT + r, ZR)],
                            agg_hbm.at[cid, pl.ds(sid * RT + r, ZR)])

    tl = tail if tail else 8
    k = pl.kernel(
        body,
        out_type=jax.ShapeDtypeStruct((NC, nnp, D), jnp.float32),
        mesh=_mesh(),
        scratch_types=[
            pltpu.VMEM((4, CH), jnp.int32),
            pltpu.VMEM((4, CH), jnp.int32),
            pltpu.VMEM((1, tl), jnp.int32),
            pltpu.VMEM((1, tl), jnp.int32),
            pltpu.VMEM((2, CH, D), jnp.float32),
            pltpu.VMEM((tl, D), jnp.float32),
            pltpu.VMEM((ZR, D), jnp.float32),
            pltpu.VMEM_SHARED((nnp, D), jnp.float32),
            pltpu.SemaphoreType.DMA,
            pltpu.SemaphoreType.DMA,
            pltpu.SemaphoreType.DMA,
            pltpu.SemaphoreType.DMA,
            pltpu.SemaphoreType.DMA,
            pltpu.SemaphoreType.DMA,
            pltpu.SemaphoreType.DMA,
            pltpu.SemaphoreType.DMA,
        ],
        compiler_params=_sc_params(),
    )
    return k(hn, src, dst)


def _tc_prolog(deg_part, x, W_in, b_in):
    nn, D = x.shape
    nnp = deg_part.shape[1]

    def body(dp_ref, x_ref, w_ref, b_ref, h_ref, nc_ref, nr_ref):
        deg = jnp.sum(dp_ref[...], axis=0, keepdims=True)
        norm_r = lax.rsqrt(jnp.maximum(deg, 1.0))
        nr_ref[...] = norm_r
        norm_c = jnp.transpose(norm_r)
        nc_ref[...] = norm_c
        h = jnp.dot(x_ref[...], w_ref[...],
                    preferred_element_type=jnp.float32, precision=_HIGH)
        h_ref[...] = (h + b_ref[...]) * norm_c[:nn]

    return pl.pallas_call(
        body,
        out_shape=[
            jax.ShapeDtypeStruct((nn, D), jnp.float32),
            jax.ShapeDtypeStruct((nnp, 1), jnp.float32),
            jax.ShapeDtypeStruct((1, nnp), jnp.float32),
        ],
    )(deg_part, x, W_in, b_in)


def _tc_mid(agg_part, norm_col, W, b, nn):
    D = agg_part.shape[2]

    def body(a_ref, nc_ref, w_ref, b_ref, o_ref):
        norm = nc_ref[...][:nn]
        a = a_ref[...]
        agg = (a[0, :nn] + a[1, :nn]) * norm
        h = jnp.dot(agg, w_ref[...],
                    preferred_element_type=jnp.float32, precision=_HIGH)
        o_ref[...] = jnp.maximum(h + b_ref[...], 0.0) * norm

    return pl.pallas_call(
        body, out_shape=jax.ShapeDtypeStruct((nn, D), jnp.float32),
    )(agg_part, norm_col, W, b)


def _tc_final(agg_part, norm_col, norm_row, w_part, W1, b1, W2, b2,
              W_out, b_out, nn):
    D = agg_part.shape[2]

    def body(a_ref, nc_ref, nr_ref, wp_ref, w1_ref, b1_ref, w2_ref, b2_ref,
             wo_ref, bo_ref, o_ref):
        norm = nc_ref[...][:nn]
        a = a_ref[...]
        agg = (a[0, :nn] + a[1, :nn]) * norm
        hc = jnp.maximum(
            jnp.dot(agg, w1_ref[...],
                    preferred_element_type=jnp.float32, precision=_HIGH)
            + b1_ref[...], 0.0)
        w_row = jnp.sum(wp_ref[...], axis=0, keepdims=True)
        c_col = jnp.transpose(w_row * nr_ref[...])[:nn]
        v = jnp.sum(hc * c_col, axis=0, keepdims=True)
        t = jnp.dot(v, w2_ref[...],
                    preferred_element_type=jnp.float32, precision=_HIGH)
        t = t + jnp.float32(nn) * b2_ref[...]
        o_ref[...] = jnp.dot(t, wo_ref[...],
                             preferred_element_type=jnp.float32,
                             precision=_HIGH) + bo_ref[...]

    return pl.pallas_call(
        body, out_shape=jax.ShapeDtypeStruct((1, D), jnp.float32),
    )(agg_part, norm_col, norm_row, w_part, W1, b1, W2, b2, W_out, b_out)


def kernel(x, edge_index, W_in, b_in, W0, b0, W1, b1, W2, b2, W_out, b_out):
    nn, D = x.shape
    nnp = _pad(nn)
    b_in2 = b_in.reshape(1, D)
    b02 = b0.reshape(1, D)
    b12 = b1.reshape(1, D)
    b22 = b2.reshape(1, D)
    b_out2 = b_out.reshape(1, D)
    src = edge_index[0]
    dst = edge_index[1]

    deg_part = _deg_sc(dst, nn).reshape(NW, nnp)
    h0n, norm_col, norm_row = _tc_prolog(deg_part, x, W_in, b_in2)
    w_part = _w_sc(src, dst, norm_row, nn).reshape(NW, nnp)
    agg0 = _agg_sc(h0n, src, dst)
    h1n = _tc_mid(agg0, norm_col, W0, b02, nn)
    agg1 = _agg_sc(h1n, src, dst)
    return _tc_final(agg1, norm_col, norm_row, w_part, W1, b12, W2, b22,
                     W_out, b_out2, nn)
